# Initial kernel scaffold; baseline (speedup 1.0000x reference)
import jax
import jax.numpy as jnp
from jax.experimental import pallas as pl


def kernel(x, segment_info, table, seg_table, pos):
    raise NotImplementedError("write your pallas kernel here")


# speedup vs baseline: 1.1200x; 1.1200x over previous
import functools

import jax
import jax.numpy as jnp
from jax import lax
from jax.experimental import pallas as pl
from jax.experimental.pallas import tpu as pltpu
from jax.experimental.pallas import tpu_sc as plsc

B, L, V, D = 1024, 200, 1000000, 64
NC, NS = 2, 16
NW = NC * NS
ROWS = B * L
RPW = ROWS // NW
CHUNK = 128
NCHUNK = RPW // CHUNK
NCOMB = 3 * L


@functools.partial(
    pl.kernel,
    out_type=jax.ShapeDtypeStruct((ROWS, D), jnp.float32),
    mesh=plsc.VectorSubcoreMesh(core_axis_name="c", subcore_axis_name="s"),
    compiler_params=pltpu.CompilerParams(use_tc_tiling_on_sc=False),
    scratch_types=[
        pltpu.VMEM((RPW,), jnp.int32),
        pltpu.VMEM((RPW,), jnp.int32),
        pltpu.VMEM((L * D,), jnp.float32),
        pltpu.VMEM((3 * D,), jnp.float32),
        pltpu.VMEM((NCOMB * D,), jnp.float32),
        pltpu.VMEM((CHUNK, D), jnp.float32),
        pltpu.SemaphoreType.DMA,
    ],
)
def _sc_embed(x_h, seg_h, table_h, segt_h, pos_h, out_h,
              idx_v, cidx_v, pos_v, segt_v, comb_v, buf_v, sem):
    cid = lax.axis_index("c")
    sid = lax.axis_index("s")
    wid = sid * NC + cid

    pltpu.sync_copy(pos_h, pos_v)
    pltpu.sync_copy(segt_h, segt_v)
    pltpu.sync_copy(x_h.at[wid], idx_v)
    pltpu.sync_copy(seg_h.at[wid], cidx_v)

    iota = lax.iota(jnp.int32, 16)

    def comb_body(l, carry):
        pbase = pl.multiple_of(l * D, D)
        cbase = pl.multiple_of(3 * l * D, D)
        for s in range(3):
            for q in range(D // 16):
                pv = pos_v[pl.ds(pbase + q * 16, 16)]
                sv = segt_v[pl.ds(s * D + q * 16, 16)]
                comb_v[pl.ds(cbase + s * D + q * 16, 16)] = pv + sv
        return carry

    lax.fori_loop(0, L, comb_body, 0)

    def cidx_body(g, carry):
        base = pl.multiple_of(g * 16, 16)
        svec = cidx_v[pl.ds(base, 16)]
        lvec = lax.rem(base + iota, jnp.full((16,), L, jnp.int32))
        cidx_v[pl.ds(base, 16)] = lvec * 3 + svec
        return carry

    lax.fori_loop(0, RPW // 16, cidx_body, 0)

    def chunk_body(k, carry):
        off = pl.multiple_of(k * CHUNK, CHUNK)
        pltpu.async_copy(table_h.at[idx_v.at[pl.ds(off, CHUNK)]], buf_v, sem).wait()

        def group_body(g, c2):
            base = pl.multiple_of(g * 16, 16)
            cvec = cidx_v[pl.ds(off + base, 16)]
            for i in range(16):
                cb = pl.multiple_of(cvec[i] * D, D)
                for q in range(D // 16):
                    av = comb_v[pl.ds(cb + q * 16, 16)]
                    buf_v[base + i, pl.ds(q * 16, 16)] += av
            return c2

        lax.fori_loop(0, CHUNK // 16, group_body, 0)
        dst = pl.multiple_of(wid * RPW + k * CHUNK, CHUNK)
        pltpu.sync_copy(buf_v, out_h.at[pl.ds(dst, CHUNK), :])
        return carry

    lax.fori_loop(0, NCHUNK, chunk_body, 0)


def kernel(x, segment_info, table, seg_table, pos):
    xf = x.reshape(NW, RPW).astype(jnp.int32)
    sf = segment_info.reshape(NW, RPW).astype(jnp.int32)
    out = _sc_embed(xf, sf, table, seg_table.reshape(-1), pos[:L].reshape(-1))
    return out.reshape(B, L, D)


# speedup vs baseline: 1.6127x; 1.4398x over previous
import functools

import jax
import jax.numpy as jnp
from jax import lax
from jax.experimental import pallas as pl
from jax.experimental.pallas import tpu as pltpu
from jax.experimental.pallas import tpu_sc as plsc

B, L, V, D = 1024, 200, 1000000, 64
NC, NS = 2, 16
NW = NC * NS
ROWS = B * L
RPW = ROWS // NW
CHUNK = 128
NCHUNK = RPW // CHUNK
NCOMB = 3 * L


@functools.partial(
    pl.kernel,
    out_type=jax.ShapeDtypeStruct((ROWS, D), jnp.float32),
    mesh=plsc.VectorSubcoreMesh(core_axis_name="c", subcore_axis_name="s"),
    scratch_types=[
        pltpu.VMEM((RPW,), jnp.int32),
        pltpu.VMEM((RPW,), jnp.int32),
        pltpu.VMEM((L * D,), jnp.float32),
        pltpu.VMEM((3 * D,), jnp.float32),
        pltpu.VMEM((NCOMB * D,), jnp.float32),
        pltpu.VMEM((CHUNK, D), jnp.float32),
        pltpu.SemaphoreType.DMA,
    ],
)
def _sc_embed(x_h, seg_h, table_h, segt_h, pos_h, out_h,
              idx_v, cidx_v, pos_v, segt_v, comb_v, buf_v, sem):
    cid = lax.axis_index("c")
    sid = lax.axis_index("s")
    wid = sid * NC + cid
    wbase = pl.multiple_of(wid * RPW, RPW)

    pltpu.sync_copy(pos_h, pos_v)
    pltpu.sync_copy(segt_h, segt_v)
    pltpu.sync_copy(x_h.at[pl.ds(wbase, RPW)], idx_v)
    pltpu.sync_copy(seg_h.at[pl.ds(wbase, RPW)], cidx_v)

    iota = lax.iota(jnp.int32, 16)

    def comb_body(l, carry):
        pbase = pl.multiple_of(l * D, D)
        cbase = pl.multiple_of(3 * l * D, D)
        for s in range(3):
            for q in range(D // 16):
                pv = pos_v[pl.ds(pbase + q * 16, 16)]
                sv = segt_v[pl.ds(s * D + q * 16, 16)]
                comb_v[pl.ds(cbase + s * D + q * 16, 16)] = pv + sv
        return carry

    lax.fori_loop(0, L, comb_body, 0)

    def cidx_body(g, carry):
        base = pl.multiple_of(g * 16, 16)
        svec = cidx_v[pl.ds(base, 16)]
        lvec = lax.rem(base + iota, jnp.full((16,), L, jnp.int32))
        cidx_v[pl.ds(base, 16)] = lvec * 3 + svec
        return carry

    lax.fori_loop(0, RPW // 16, cidx_body, 0)

    def chunk_body(k, carry):
        off = pl.multiple_of(k * CHUNK, CHUNK)

        def fetch_body(g, c2):
            gb = pl.multiple_of(g * 16, 16)
            ivec = idx_v[pl.ds(off + gb, 16)]
            for i in range(16):
                pltpu.async_copy(table_h.at[ivec[i]], buf_v.at[gb + i], sem)
            return c2

        lax.fori_loop(0, CHUNK // 16, fetch_body, 0)
        pltpu.make_async_copy(out_h.at[pl.ds(0, CHUNK), :], buf_v, sem).wait()

        def group_body(g, c2):
            base = pl.multiple_of(g * 16, 16)
            cvec = cidx_v[pl.ds(off + base, 16)]
            for i in range(16):
                cb = pl.multiple_of(cvec[i] * D, D)
                for q in range(D // 16):
                    av = comb_v[pl.ds(cb + q * 16, 16)]
                    buf_v[base + i, pl.ds(q * 16, 16)] += av
            return c2

        lax.fori_loop(0, CHUNK // 16, group_body, 0)
        dst = pl.multiple_of(wbase + k * CHUNK, CHUNK)
        pltpu.sync_copy(buf_v, out_h.at[pl.ds(dst, CHUNK), :])
        return carry

    lax.fori_loop(0, NCHUNK, chunk_body, 0)


def kernel(x, segment_info, table, seg_table, pos):
    xf = x.reshape(-1).astype(jnp.int32)
    sf = segment_info.reshape(-1).astype(jnp.int32)
    out = _sc_embed(xf, sf, table, seg_table.reshape(-1), pos[:L].reshape(-1))
    return out.reshape(B, L, D)


# speedup vs baseline: 1.7241x; 1.0691x over previous
import functools

import jax
import jax.numpy as jnp
from jax import lax
from jax.experimental import pallas as pl
from jax.experimental.pallas import tpu as pltpu
from jax.experimental.pallas import tpu_sc as plsc

B, L, V, D = 1024, 200, 1000000, 64
NC, NS = 2, 16
NW = NC * NS
BPW = B // NW
NCOMB = 3 * L
NWIN = L // 16


@functools.partial(
    pl.kernel,
    out_type=jax.ShapeDtypeStruct((B * L, D), jnp.float32),
    mesh=plsc.VectorSubcoreMesh(core_axis_name="c", subcore_axis_name="s"),
    scratch_types=[
        pltpu.VMEM((BPW, L), jnp.int32),
        pltpu.VMEM((BPW, L), jnp.int32),
        pltpu.VMEM((3, D), jnp.float32),
        pltpu.VMEM((NCOMB * D,), jnp.float32),
        pltpu.VMEM((L, D), jnp.float32),
        pltpu.VMEM((L, D), jnp.float32),
        pltpu.SMEM((L,), jnp.int32),
        pltpu.SMEM((L,), jnp.int32),
        pltpu.SemaphoreType.DMA,
        pltpu.SemaphoreType.DMA,
        pltpu.SemaphoreType.DMA,
        pltpu.SemaphoreType.DMA,
    ],
)
def _sc_embed(x_h, seg_h, table_h, segt_h, pos_h, out_h,
              xv, sv, segt_v, comb_v, buf0, buf1, sm0, sm1,
              gsem0, gsem1, osem0, osem1):
    cid = lax.axis_index("c")
    sid = lax.axis_index("s")
    wid = sid * NC + cid
    bbase = pl.multiple_of(wid * BPW, BPW)

    pltpu.sync_copy(x_h.at[pl.ds(bbase, BPW), :], xv)
    pltpu.sync_copy(seg_h.at[pl.ds(bbase, BPW), :], sv)
    pltpu.sync_copy(segt_h, segt_v)
    pltpu.sync_copy(pos_h.at[pl.ds(0, L), :], buf0)

    def comb_body(l, carry):
        cbase = pl.multiple_of(3 * l * D, D)
        for s in range(3):
            for q in range(D // 16):
                pv = buf0[l, pl.ds(q * 16, 16)]
                tv = segt_v[s, pl.ds(q * 16, 16)]
                comb_v[pl.ds(cbase + s * D + q * 16, 16)] = pv + tv
        return carry

    lax.fori_loop(0, L, comb_body, 0)

    bufs = (buf0, buf1)
    stash = (sm0, sm1)
    gsems = (gsem0, gsem1)
    osems = (osem0, osem1)

    def fetch(bb, slot):
        buf, sm = bufs[slot], stash[slot]
        for w in range(NWIN + 1):
            wstart = w * 16 if w < NWIN else L - 16
            lanes = range(16) if w < NWIN else range(8, 16)
            xvec = xv[bb, pl.ds(wstart, 16)]
            svec = sv[bb, pl.ds(wstart, 16)]
            pvec = xvec * 4 + svec
            for i in lanes:
                p = pvec[i]
                j = wstart + i
                pltpu.async_copy(table_h.at[p >> 2], buf.at[j], gsems[slot])
                sm[j] = p & 3

    def process(bb, slot):
        buf, sm = bufs[slot], stash[slot]
        pltpu.make_async_copy(out_h.at[pl.ds(0, L), :], buf, gsems[slot]).wait()

        def add_body(j, carry):
            aoff = pl.multiple_of((3 * j + sm[j]) * D, D)
            for q in range(D // 16):
                av = comb_v[pl.ds(aoff + q * 16, 16)]
                buf[j, pl.ds(q * 16, 16)] += av
            return carry

        lax.fori_loop(0, L, add_body, 0)
        dst = pl.multiple_of((bbase + bb) * L, L)
        pltpu.async_copy(buf, out_h.at[pl.ds(dst, L), :], osems[slot])

    def drain_out(slot):
        pltpu.make_async_copy(bufs[slot], out_h.at[pl.ds(0, L), :],
                              osems[slot]).wait()

    fetch(0, 0)

    def pair_body(h, carry):
        bb = h * 2

        @pl.when(h > 0)
        def _():
            drain_out(1)
        fetch(bb + 1, 1)
        process(bb, 0)

        drain_out(0)

        @pl.when(h + 1 < BPW // 2)
        def _():
            fetch(bb + 2, 0)
        process(bb + 1, 1)
        return carry

    lax.fori_loop(0, BPW // 2, pair_body, 0)
    drain_out(1)


def kernel(x, segment_info, table, seg_table, pos):
    out = _sc_embed(x.astype(jnp.int32), segment_info.astype(jnp.int32),
                    table, seg_table, pos)
    return out.reshape(B, L, D)
